# baseline (speedup 1.0000x reference)
import functools

import jax
import jax.numpy as jnp
from jax.experimental import pallas as pl
from jax.experimental.pallas import tpu as pltpu

_BATCH_TILE = 8


def _freq_kernel(alf_ref, gf_ref, his_ref, out_ref, g_scratch):
    @pl.when(pl.program_id(0) == 0)
    def _():
        row = gf_ref[...]
        m = jnp.max(row)
        e = jnp.exp(row - m)
        g_scratch[...] = e / jnp.sum(e)

    a = jax.nn.sigmoid(alf_ref[0])
    out_ref[...] = a * his_ref[...] + (1.0 - a) * g_scratch[...]


def kernel(his, global_freq_table, alf):
    batch, num_items = his.shape
    grid = (batch // _BATCH_TILE,)
    return pl.pallas_call(
        _freq_kernel,
        grid=grid,
        in_specs=[
            pl.BlockSpec(memory_space=pltpu.SMEM),
            pl.BlockSpec((1, num_items), lambda i: (0, 0)),
            pl.BlockSpec((_BATCH_TILE, num_items), lambda i: (i, 0)),
        ],
        out_specs=pl.BlockSpec((_BATCH_TILE, num_items), lambda i: (i, 0)),
        out_shape=jax.ShapeDtypeStruct((batch, num_items), his.dtype),
        scratch_shapes=[pltpu.VMEM((1, num_items), jnp.float32)],
    )(alf, global_freq_table, his)


# speedup vs baseline: 1.0146x; 1.0146x over previous
import jax
import jax.numpy as jnp
from jax.experimental import pallas as pl
from jax.experimental.pallas import tpu as pltpu

_BATCH_TILE = 8


def _freq_kernel(alf_ref, gf_ref, his_ref, out_ref, g_scratch):
    a = jax.nn.sigmoid(alf_ref[0])

    @pl.when(pl.program_id(0) == 0)
    def _():
        row = gf_ref[...]
        m = jnp.max(row)
        e = jnp.exp(row - m)
        g = (1.0 - a) * (e / jnp.sum(e))
        g_scratch[...] = jnp.broadcast_to(g, g_scratch.shape)

    out_ref[...] = a * his_ref[...] + g_scratch[...]


def kernel(his, global_freq_table, alf):
    batch, num_items = his.shape
    grid = (batch // _BATCH_TILE,)
    return pl.pallas_call(
        _freq_kernel,
        grid=grid,
        in_specs=[
            pl.BlockSpec(memory_space=pltpu.SMEM),
            pl.BlockSpec((1, num_items), lambda i: (0, 0)),
            pl.BlockSpec((_BATCH_TILE, num_items), lambda i: (i, 0)),
        ],
        out_specs=pl.BlockSpec((_BATCH_TILE, num_items), lambda i: (i, 0)),
        out_shape=jax.ShapeDtypeStruct((batch, num_items), his.dtype),
        scratch_shapes=[pltpu.VMEM((_BATCH_TILE, num_items), jnp.float32)],
    )(alf, global_freq_table, his)


# speedup vs baseline: 1.0192x; 1.0045x over previous
import jax
import jax.numpy as jnp
from jax.experimental import pallas as pl
from jax.experimental.pallas import tpu as pltpu

_BATCH_TILE = 16


def _freq_kernel(alf_ref, gf_ref, his_ref, out_ref, g_scratch):
    a = jax.nn.sigmoid(alf_ref[0])

    @pl.when(pl.program_id(0) == 0)
    def _():
        row = gf_ref[...]
        m = jnp.max(row)
        e = jnp.exp(row - m)
        g = (1.0 - a) * (e / jnp.sum(e))
        g_scratch[...] = jnp.broadcast_to(g, g_scratch.shape)

    out_ref[...] = a * his_ref[...] + g_scratch[...]


def kernel(his, global_freq_table, alf):
    batch, num_items = his.shape
    grid = (batch // _BATCH_TILE,)
    return pl.pallas_call(
        _freq_kernel,
        grid=grid,
        in_specs=[
            pl.BlockSpec(memory_space=pltpu.SMEM),
            pl.BlockSpec((1, num_items), lambda i: (0, 0)),
            pl.BlockSpec((_BATCH_TILE, num_items), lambda i: (i, 0)),
        ],
        out_specs=pl.BlockSpec((_BATCH_TILE, num_items), lambda i: (i, 0)),
        out_shape=jax.ShapeDtypeStruct((batch, num_items), his.dtype),
        scratch_shapes=[pltpu.VMEM((_BATCH_TILE, num_items), jnp.float32)],
    )(alf, global_freq_table, his)
